# baseline (speedup 1.0000x reference)
import functools

import jax
import jax.numpy as jnp
from jax import lax
from jax.experimental import pallas as pl
from jax.experimental.pallas import tpu as pltpu
from jax.experimental.pallas import tpu_sc as plsc

_B = 131072
_K = 512
_D = 64
_COMMIT = 0.9

_NSEG = 4
_BSEG = _B // _NSEG

_BLK = 4096
_NBLK = _BSEG // _BLK


def _argmin_body(x_ref, cb_ref, idx_ref, loss_ref):
    i = pl.program_id(0)
    x = x_ref[...]
    cb = cb_ref[...]
    ones_row = jnp.ones((1, _D), jnp.float32)
    x2 = lax.dot_general(ones_row, x * x, (((1,), (1,)), ((), ())),
                         preferred_element_type=jnp.float32)
    e2 = lax.dot_general(cb * cb, ones_row, (((1,), (1,)), ((), ())),
                         preferred_element_type=jnp.float32)
    mm = lax.dot_general(cb, x, (((1,), (1,)), ((), ())),
                         preferred_element_type=jnp.float32)
    dist = (x2 + e2) - 2.0 * mm
    minv = jnp.min(dist, axis=0, keepdims=True)
    ks = lax.broadcasted_iota(jnp.int32, (_K, _BLK), 0)
    idx = jnp.min(jnp.where(dist == minv, ks, _K), axis=0)
    idx_ref[...] = idx
    s = jnp.sum(minv)

    @pl.when(i == 0)
    def _():
        loss_ref[0, 0] = s

    @pl.when(i > 0)
    def _():
        loss_ref[0, 0] += s


_argmin_call = pl.pallas_call(
    _argmin_body,
    grid=(_NBLK,),
    in_specs=[
        pl.BlockSpec((_BLK, _D), lambda i: (i, 0)),
        pl.BlockSpec((_K, _D), lambda i: (0, 0)),
    ],
    out_specs=[
        pl.BlockSpec((_BLK,), lambda i: (i,)),
        pl.BlockSpec(memory_space=pltpu.SMEM),
    ],
    out_shape=[
        jax.ShapeDtypeStruct((_BSEG,), jnp.int32),
        jax.ShapeDtypeStruct((1, 1), jnp.float32),
    ],
)

_NUM_SC_CORES = 2
_NUM_SC_SUBCORES = 16
_NW = _NUM_SC_CORES * _NUM_SC_SUBCORES
_BPW = _BSEG // _NW
_CH = 128
_NCH = _BPW // _CH


def _sc_gather_body(cb_hbm, idx_hbm, out_hbm, idx_v, rows0, rows1, sem0, sem1):
    wid = lax.axis_index("s") * _NUM_SC_CORES + lax.axis_index("c")
    base = wid * _BPW
    pltpu.sync_copy(idx_hbm.at[pl.ds(base, _BPW)], idx_v)

    def fire(j, buf, sem):
        return pltpu.async_copy(cb_hbm.at[idx_v.at[pl.ds(j * _CH, _CH)]],
                                buf, sem)

    fire(0, rows0, sem0)

    def body(p, carry):
        j = 2 * p
        fire(j + 1, rows1, sem1)
        pltpu.make_async_copy(cb_hbm.at[idx_v.at[pl.ds(j * _CH, _CH)]],
                              rows0, sem0).wait()
        pltpu.sync_copy(rows0, out_hbm.at[pl.ds(base + j * _CH, _CH)])

        @pl.when(j + 2 < _NCH)
        def _():
            fire(j + 2, rows0, sem0)

        pltpu.make_async_copy(cb_hbm.at[idx_v.at[pl.ds((j + 1) * _CH, _CH)]],
                              rows1, sem1).wait()
        pltpu.sync_copy(rows1, out_hbm.at[pl.ds(base + (j + 1) * _CH, _CH)])
        return carry

    lax.fori_loop(0, _NCH // 2, body, 0)


@functools.cache
def _sc_gather_call():
    return pl.kernel(
        _sc_gather_body,
        mesh=plsc.VectorSubcoreMesh(core_axis_name="c", subcore_axis_name="s"),
        out_type=jax.ShapeDtypeStruct((_BSEG, _D), jnp.float32),
        scratch_types=[
            pltpu.VMEM((_BPW,), jnp.int32),
            pltpu.VMEM((_CH, _D), jnp.float32),
            pltpu.VMEM((_CH, _D), jnp.float32),
            pltpu.SemaphoreType.DMA,
            pltpu.SemaphoreType.DMA,
        ],
        compiler_params=pltpu.CompilerParams(use_tc_tiling_on_sc=False),
    )


def kernel(inputs, codebook):
    sc = _sc_gather_call()
    idx_parts, q_parts, loss_sum = [], [], None
    for seg in range(_NSEG):
        xs = lax.slice_in_dim(inputs, seg * _BSEG, (seg + 1) * _BSEG, axis=0)
        idx_s, loss_s = _argmin_call(xs, codebook)
        q_parts.append(sc(codebook, idx_s))
        idx_parts.append(idx_s)
        loss_sum = loss_s if loss_sum is None else loss_sum + loss_s
    quantized = jnp.concatenate(q_parts, axis=0)
    idx = jnp.concatenate(idx_parts, axis=0)
    loss = loss_sum.reshape(()) * ((1.0 + _COMMIT) / (_B * _D))
    return quantized, loss, idx


# speedup vs baseline: 1.9610x; 1.9610x over previous
import functools

import jax
import jax.numpy as jnp
from jax import lax
from jax.experimental import pallas as pl
from jax.experimental.pallas import tpu as pltpu
from jax.experimental.pallas import tpu_sc as plsc

_B = 131072
_K = 512
_D = 64
_COMMIT = 0.9

_BLK = 4096
_NBLK = _B // _BLK


def _argmin_body(xt_ref, cb_ref, idx_ref, loss_ref):
    i = pl.program_id(0)
    xt = xt_ref[...]
    cb = cb_ref[...]
    ones_row = jnp.ones((1, _D), jnp.float32)
    x2 = lax.dot_general(ones_row, xt * xt, (((1,), (0,)), ((), ())),
                         preferred_element_type=jnp.float32)
    e2 = lax.dot_general(cb * cb, ones_row, (((1,), (1,)), ((), ())),
                         preferred_element_type=jnp.float32)
    mm = lax.dot_general(cb, xt, (((1,), (0,)), ((), ())),
                         preferred_element_type=jnp.float32)
    dist = (x2 + e2) - 2.0 * mm
    minv = jnp.min(dist, axis=0, keepdims=True)
    ks = lax.broadcasted_iota(jnp.int32, (_K, _BLK), 0)
    idx = jnp.min(jnp.where(dist == minv, ks, _K), axis=0)
    idx_ref[...] = idx
    s = jnp.sum(minv)

    @pl.when(i == 0)
    def _():
        loss_ref[0, 0] = s

    @pl.when(i > 0)
    def _():
        loss_ref[0, 0] += s

    @pl.when(i == _NBLK - 1)
    def _():
        loss_ref[0, 0] = loss_ref[0, 0] * ((1.0 + _COMMIT) / (_B * _D))


_argmin_call = pl.pallas_call(
    _argmin_body,
    grid=(_NBLK,),
    in_specs=[
        pl.BlockSpec((_D, _BLK), lambda i: (0, i)),
        pl.BlockSpec((_K, _D), lambda i: (0, 0)),
    ],
    out_specs=[
        pl.BlockSpec((_BLK,), lambda i: (i,)),
        pl.BlockSpec(memory_space=pltpu.SMEM),
    ],
    out_shape=[
        jax.ShapeDtypeStruct((_B,), jnp.int32),
        jax.ShapeDtypeStruct((1, 1), jnp.float32),
    ],
)

_NUM_SC_CORES = 2
_NUM_SC_SUBCORES = 16
_NW = _NUM_SC_CORES * _NUM_SC_SUBCORES
_BPW = _B // _NW
_CCH = 512
_NCCH = _BPW // _CCH


def _sc_gather_body(cbt_hbm, idx_hbm, qt_hbm, cbt_v, idx_v, qt_v):
    wid = lax.axis_index("s") * _NUM_SC_CORES + lax.axis_index("c")
    base = wid * _BPW
    pltpu.sync_copy(cbt_hbm, cbt_v)
    pltpu.sync_copy(idx_hbm.at[pl.ds(base, _BPW)], idx_v)
    dvecs = [jnp.full((16,), d, jnp.int32) for d in range(_D)]

    for c in range(_NCCH):
        def gbody(g, carry, c=c):
            iv = idx_v[pl.ds(c * _CCH + g * 16, 16)]
            for d in range(_D):
                qt_v[d, pl.ds(g * 16, 16)] = plsc.load_gather(
                    cbt_v, [dvecs[d], iv])
            return carry

        lax.fori_loop(0, _CCH // 16, gbody, 0)
        pltpu.sync_copy(qt_v, qt_hbm.at[:, pl.ds(base + c * _CCH, _CCH)])


@functools.cache
def _sc_gather_call():
    return pl.kernel(
        _sc_gather_body,
        mesh=plsc.VectorSubcoreMesh(core_axis_name="c", subcore_axis_name="s"),
        out_type=jax.ShapeDtypeStruct((_D, _B), jnp.float32),
        scratch_types=[
            pltpu.VMEM((_D, _K), jnp.float32),
            pltpu.VMEM((_BPW,), jnp.int32),
            pltpu.VMEM((_D, _CCH), jnp.float32),
        ],
        compiler_params=pltpu.CompilerParams(needs_layout_passes=False),
    )


def kernel(inputs, codebook):
    xt = inputs.T
    idx, loss = _argmin_call(xt, codebook)
    qt = _sc_gather_call()(codebook.T, idx)
    return qt.T, loss.reshape(()), idx


# speedup vs baseline: 2.5787x; 1.3150x over previous
import functools

import jax
import jax.numpy as jnp
from jax import lax
from jax.experimental import pallas as pl
from jax.experimental.pallas import tpu as pltpu
from jax.experimental.pallas import tpu_sc as plsc

_B = 131072
_K = 512
_D = 64
_COMMIT = 0.9

_BLK = 4096
_NBLK = _B // _BLK


def _argmin_body(xt_ref, cb_ref, idx_ref, loss_ref):
    i = pl.program_id(0)
    xt = xt_ref[...]
    cb = cb_ref[...]
    ones_row = jnp.ones((1, _D), jnp.float32)
    x2 = lax.dot_general(ones_row, xt * xt, (((1,), (0,)), ((), ())),
                         preferred_element_type=jnp.float32)
    e2 = lax.dot_general(cb * cb, ones_row, (((1,), (1,)), ((), ())),
                         preferred_element_type=jnp.float32)
    mm = lax.dot_general(cb, xt, (((1,), (0,)), ((), ())),
                         preferred_element_type=jnp.float32)
    dist = (x2 + e2) - 2.0 * mm
    minv = jnp.min(dist, axis=0, keepdims=True)
    ks = lax.broadcasted_iota(jnp.int32, (_K, _BLK), 0)
    idx = jnp.min(jnp.where(dist == minv, ks, _K), axis=0)
    idx_ref[...] = idx
    s = jnp.sum(minv)

    @pl.when(i == 0)
    def _():
        loss_ref[0, 0] = s

    @pl.when(i > 0)
    def _():
        loss_ref[0, 0] += s

    @pl.when(i == _NBLK - 1)
    def _():
        loss_ref[0, 0] = loss_ref[0, 0] * ((1.0 + _COMMIT) / (_B * _D))


_argmin_call = pl.pallas_call(
    _argmin_body,
    grid=(_NBLK,),
    in_specs=[
        pl.BlockSpec((_D, _BLK), lambda i: (0, i)),
        pl.BlockSpec((_K, _D), lambda i: (0, 0)),
    ],
    out_specs=[
        pl.BlockSpec((_BLK,), lambda i: (i,)),
        pl.BlockSpec(memory_space=pltpu.SMEM),
    ],
    out_shape=[
        jax.ShapeDtypeStruct((_B,), jnp.int32),
        jax.ShapeDtypeStruct((1, 1), jnp.float32),
    ],
)

_NUM_SC_CORES = 2
_NUM_SC_SUBCORES = 16
_NW = _NUM_SC_CORES * _NUM_SC_SUBCORES
_BPW = _B // _NW
_CCH = 512
_NCCH = _BPW // _CCH


_DB = 8


def _sc_gather_body(cbt_hbm, idx_hbm, qt_hbm, cbt_v, idx_v, qt0, qt1,
                    sem0, sem1):
    wid = lax.axis_index("s") * _NUM_SC_CORES + lax.axis_index("c")
    base = wid * _BPW
    pltpu.sync_copy(cbt_hbm, cbt_v)
    pltpu.sync_copy(idx_hbm.at[pl.ds(base, _BPW)], idx_v)
    dvecs = [jnp.full((16,), d, jnp.int32) for d in range(_D)]
    bufs = (qt0, qt1)
    sems = (sem0, sem1)

    for c in range(_NCCH):
        buf, sem = bufs[c % 2], sems[c % 2]
        if c >= 2:
            pltpu.make_async_copy(
                buf, qt_hbm.at[:, pl.ds(base + (c - 2) * _CCH, _CCH)],
                sem).wait()

        def gbody(g, carry, buf=buf, c=c):
            iv = idx_v[pl.ds(c * _CCH + g * 16, 16)]
            for d0 in range(0, _D, _DB):
                vals = [plsc.load_gather(cbt_v, [dvecs[d0 + u], iv])
                        for u in range(_DB)]
                for u in range(_DB):
                    buf[d0 + u, pl.ds(g * 16, 16)] = vals[u]
            return carry

        lax.fori_loop(0, _CCH // 16, gbody, 0)
        pltpu.async_copy(buf, qt_hbm.at[:, pl.ds(base + c * _CCH, _CCH)], sem)

    for c in (_NCCH - 2, _NCCH - 1):
        pltpu.make_async_copy(
            bufs[c % 2], qt_hbm.at[:, pl.ds(base + c * _CCH, _CCH)],
            sems[c % 2]).wait()


@functools.cache
def _sc_gather_call():
    return pl.kernel(
        _sc_gather_body,
        mesh=plsc.VectorSubcoreMesh(core_axis_name="c", subcore_axis_name="s"),
        out_type=jax.ShapeDtypeStruct((_D, _B), jnp.float32),
        scratch_types=[
            pltpu.VMEM((_D, _K), jnp.float32),
            pltpu.VMEM((_BPW,), jnp.int32),
            pltpu.VMEM((_D, _CCH), jnp.float32),
            pltpu.VMEM((_D, _CCH), jnp.float32),
            pltpu.SemaphoreType.DMA,
            pltpu.SemaphoreType.DMA,
        ],
        compiler_params=pltpu.CompilerParams(needs_layout_passes=False),
    )


def kernel(inputs, codebook):
    xt = inputs.T
    idx, loss = _argmin_call(xt, codebook)
    qt = _sc_gather_call()(codebook.T, idx)
    return qt.T, loss.reshape(()), idx


# speedup vs baseline: 2.6316x; 1.0205x over previous
import functools

import jax
import jax.numpy as jnp
from jax import lax
from jax.experimental import pallas as pl
from jax.experimental.pallas import tpu as pltpu
from jax.experimental.pallas import tpu_sc as plsc

_B = 131072
_K = 512
_D = 64
_COMMIT = 0.9

_BLK = 8192
_NBLK = _B // _BLK


def _argmin_body(xt_ref, cb_ref, idx_ref, loss_ref):
    i = pl.program_id(0)
    xt = xt_ref[...]
    cb = cb_ref[...]
    ones_row = jnp.ones((1, _D), jnp.float32)
    x2 = lax.dot_general(ones_row, xt * xt, (((1,), (0,)), ((), ())),
                         preferred_element_type=jnp.float32)
    e2 = lax.dot_general(cb * cb, ones_row, (((1,), (1,)), ((), ())),
                         preferred_element_type=jnp.float32)
    mm = lax.dot_general(cb, xt, (((1,), (0,)), ((), ())),
                         preferred_element_type=jnp.float32)
    dist = (x2 + e2) - 2.0 * mm
    minv = jnp.min(dist, axis=0, keepdims=True)
    ks = lax.broadcasted_iota(jnp.int32, (_K, _BLK), 0)
    idx = jnp.min(jnp.where(dist == minv, ks, _K), axis=0)
    idx_ref[...] = idx
    s = jnp.sum(minv)

    @pl.when(i == 0)
    def _():
        loss_ref[0, 0] = s

    @pl.when(i > 0)
    def _():
        loss_ref[0, 0] += s

    @pl.when(i == _NBLK - 1)
    def _():
        loss_ref[0, 0] = loss_ref[0, 0] * ((1.0 + _COMMIT) / (_B * _D))


_argmin_call = pl.pallas_call(
    _argmin_body,
    grid=(_NBLK,),
    in_specs=[
        pl.BlockSpec((_D, _BLK), lambda i: (0, i)),
        pl.BlockSpec((_K, _D), lambda i: (0, 0)),
    ],
    out_specs=[
        pl.BlockSpec((_BLK,), lambda i: (i,)),
        pl.BlockSpec(memory_space=pltpu.SMEM),
    ],
    out_shape=[
        jax.ShapeDtypeStruct((_B,), jnp.int32),
        jax.ShapeDtypeStruct((1, 1), jnp.float32),
    ],
)

_NUM_SC_CORES = 2
_NUM_SC_SUBCORES = 16
_NW = _NUM_SC_CORES * _NUM_SC_SUBCORES
_BPW = _B // _NW
_CCH = 512
_NCCH = _BPW // _CCH


_DB = 8


def _sc_gather_body(cbt_hbm, idx_hbm, qt_hbm, cbt_v, idx_v, qt0, qt1,
                    sem0, sem1):
    wid = lax.axis_index("s") * _NUM_SC_CORES + lax.axis_index("c")
    base = wid * _BPW
    pltpu.sync_copy(cbt_hbm, cbt_v)
    pltpu.sync_copy(idx_hbm.at[pl.ds(base, _BPW)], idx_v)
    dvecs = [jnp.full((16,), d, jnp.int32) for d in range(_D)]
    bufs = (qt0, qt1)
    sems = (sem0, sem1)

    for c in range(_NCCH):
        buf, sem = bufs[c % 2], sems[c % 2]
        if c >= 2:
            pltpu.make_async_copy(
                buf, qt_hbm.at[:, pl.ds(base + (c - 2) * _CCH, _CCH)],
                sem).wait()

        def gbody(g, carry, buf=buf, c=c):
            iv = idx_v[pl.ds(c * _CCH + g * 16, 16)]
            for d0 in range(0, _D, _DB):
                vals = [plsc.load_gather(cbt_v, [dvecs[d0 + u], iv])
                        for u in range(_DB)]
                for u in range(_DB):
                    buf[d0 + u, pl.ds(g * 16, 16)] = vals[u]
            return carry

        lax.fori_loop(0, _CCH // 16, gbody, 0)
        pltpu.async_copy(buf, qt_hbm.at[:, pl.ds(base + c * _CCH, _CCH)], sem)

    for c in (_NCCH - 2, _NCCH - 1):
        pltpu.make_async_copy(
            bufs[c % 2], qt_hbm.at[:, pl.ds(base + c * _CCH, _CCH)],
            sems[c % 2]).wait()


@functools.cache
def _sc_gather_call():
    return pl.kernel(
        _sc_gather_body,
        mesh=plsc.VectorSubcoreMesh(core_axis_name="c", subcore_axis_name="s"),
        out_type=jax.ShapeDtypeStruct((_D, _B), jnp.float32),
        scratch_types=[
            pltpu.VMEM((_D, _K), jnp.float32),
            pltpu.VMEM((_BPW,), jnp.int32),
            pltpu.VMEM((_D, _CCH), jnp.float32),
            pltpu.VMEM((_D, _CCH), jnp.float32),
            pltpu.SemaphoreType.DMA,
            pltpu.SemaphoreType.DMA,
        ],
        compiler_params=pltpu.CompilerParams(needs_layout_passes=False),
    )


def kernel(inputs, codebook):
    xt = inputs.T
    idx, loss = _argmin_call(xt, codebook)
    qt = _sc_gather_call()(codebook.T, idx)
    return qt.T, loss.reshape(()), idx


# speedup vs baseline: 2.6405x; 1.0034x over previous
import functools

import jax
import jax.numpy as jnp
from jax import lax
from jax.experimental import pallas as pl
from jax.experimental.pallas import tpu as pltpu
from jax.experimental.pallas import tpu_sc as plsc

_B = 131072
_K = 512
_D = 64
_COMMIT = 0.9

_BLK = 8192
_NBLK = _B // _BLK


def _argmin_body(xt_ref, cb_ref, idx_ref, loss_ref):
    i = pl.program_id(0)
    xt = xt_ref[...]
    cb = cb_ref[...]
    ones_row = jnp.ones((1, _D), jnp.float32)
    x2 = lax.dot_general(ones_row, xt * xt, (((1,), (0,)), ((), ())),
                         preferred_element_type=jnp.float32)
    e2 = lax.dot_general(cb * cb, ones_row, (((1,), (1,)), ((), ())),
                         preferred_element_type=jnp.float32)
    mm = lax.dot_general(cb, xt, (((1,), (0,)), ((), ())),
                         preferred_element_type=jnp.float32)
    dist = (x2 + e2) - 2.0 * mm
    minv = jnp.min(dist, axis=0, keepdims=True)
    ks = lax.broadcasted_iota(jnp.int32, (_K, _BLK), 0)
    idx = jnp.min(jnp.where(dist == minv, ks, _K), axis=0)
    idx_ref[...] = idx
    s = jnp.sum(minv)

    @pl.when(i == 0)
    def _():
        loss_ref[0, 0] = s

    @pl.when(i > 0)
    def _():
        loss_ref[0, 0] += s

    @pl.when(i == _NBLK - 1)
    def _():
        loss_ref[0, 0] = loss_ref[0, 0] * ((1.0 + _COMMIT) / (_B * _D))


_argmin_call = pl.pallas_call(
    _argmin_body,
    grid=(_NBLK,),
    in_specs=[
        pl.BlockSpec((_D, _BLK), lambda i: (0, i)),
        pl.BlockSpec((_K, _D), lambda i: (0, 0)),
    ],
    out_specs=[
        pl.BlockSpec((_BLK,), lambda i: (i,)),
        pl.BlockSpec(memory_space=pltpu.SMEM),
    ],
    out_shape=[
        jax.ShapeDtypeStruct((_B,), jnp.int32),
        jax.ShapeDtypeStruct((1, 1), jnp.float32),
    ],
    compiler_params=pltpu.CompilerParams(vmem_limit_bytes=117_440_512),
)

_NUM_SC_CORES = 2
_NUM_SC_SUBCORES = 16
_NW = _NUM_SC_CORES * _NUM_SC_SUBCORES
_BPW = _B // _NW
_CCH = 512
_NCCH = _BPW // _CCH


_DB = 8


def _sc_gather_body(cbt_hbm, idx_hbm, qt_hbm, cbt_v, idx_v, qt0, qt1,
                    sem0, sem1):
    wid = lax.axis_index("s") * _NUM_SC_CORES + lax.axis_index("c")
    base = wid * _BPW
    pltpu.sync_copy(cbt_hbm, cbt_v)
    pltpu.sync_copy(idx_hbm.at[pl.ds(base, _BPW)], idx_v)
    dvecs = [jnp.full((16,), d, jnp.int32) for d in range(_D)]
    bufs = (qt0, qt1)
    sems = (sem0, sem1)

    for c in range(_NCCH):
        buf, sem = bufs[c % 2], sems[c % 2]
        if c >= 2:
            pltpu.make_async_copy(
                buf, qt_hbm.at[:, pl.ds(base + (c - 2) * _CCH, _CCH)],
                sem).wait()

        def gbody(g, carry, buf=buf, c=c):
            iv = idx_v[pl.ds(c * _CCH + g * 16, 16)]
            for d0 in range(0, _D, _DB):
                vals = [plsc.load_gather(cbt_v, [dvecs[d0 + u], iv])
                        for u in range(_DB)]
                for u in range(_DB):
                    buf[d0 + u, pl.ds(g * 16, 16)] = vals[u]
            return carry

        lax.fori_loop(0, _CCH // 16, gbody, 0)
        pltpu.async_copy(buf, qt_hbm.at[:, pl.ds(base + c * _CCH, _CCH)], sem)

    for c in (_NCCH - 2, _NCCH - 1):
        pltpu.make_async_copy(
            bufs[c % 2], qt_hbm.at[:, pl.ds(base + c * _CCH, _CCH)],
            sems[c % 2]).wait()


@functools.cache
def _sc_gather_call():
    return pl.kernel(
        _sc_gather_body,
        mesh=plsc.VectorSubcoreMesh(core_axis_name="c", subcore_axis_name="s"),
        out_type=jax.ShapeDtypeStruct((_D, _B), jnp.float32),
        scratch_types=[
            pltpu.VMEM((_D, _K), jnp.float32),
            pltpu.VMEM((_BPW,), jnp.int32),
            pltpu.VMEM((_D, _CCH), jnp.float32),
            pltpu.VMEM((_D, _CCH), jnp.float32),
            pltpu.SemaphoreType.DMA,
            pltpu.SemaphoreType.DMA,
        ],
        compiler_params=pltpu.CompilerParams(needs_layout_passes=False),
    )


def kernel(inputs, codebook):
    xt = inputs.T
    idx, loss = _argmin_call(xt, codebook)
    qt = _sc_gather_call()(codebook.T, idx)
    return qt.T, loss.reshape(()), idx
